# baseline (speedup 1.0000x reference)
import functools

import jax
import jax.numpy as jnp
from jax import lax
from jax.experimental import pallas as pl
from jax.experimental.pallas import tpu as pltpu
from jax.experimental.pallas import tpu_sc as plsc


_B, _S, _D = 4, 196, 1024
_H, _K, _Dh = 8, 8192, 128
_T = _B * _S
_KC = 1024
_NKC = _K // _KC
_DA = _Dh + 8

_NW = 32
_BPW = 256
_TPAD = _NW * _BPW


def _argmin_head_kernel(enc_ref, cb_ref, idx_ref, fidx_ref, loss_ref):
    h = pl.program_id(0)
    z = enc_ref[...].reshape(_T, _Dh)
    cb = cb_ref[0]

    z_sq = jnp.sum(z * z, axis=1, keepdims=True)

    run_min = jnp.full((_T, 1), jnp.inf, dtype=jnp.float32)
    run_idx = jnp.zeros((_T, 1), dtype=jnp.int32)
    iota = jax.lax.broadcasted_iota(jnp.int32, (_T, _KC), 1)

    for kb in range(_NKC):
        cbk = cb[kb * _KC:(kb + 1) * _KC, :]
        dots = jax.lax.dot_general(
            z, cbk, (((1,), (1,)), ((), ())),
            preferred_element_type=jnp.float32)
        csq = jnp.sum(cbk * cbk, axis=1)[None, :]
        d = z_sq - 2.0 * dots + csq
        mv = jnp.min(d, axis=1, keepdims=True)
        cand = jnp.where(d == mv, iota, _K)
        li = jnp.min(cand, axis=1, keepdims=True).astype(jnp.int32)
        better = mv < run_min
        run_idx = jnp.where(better, li + kb * _KC, run_idx)
        run_min = jnp.where(better, mv, run_min)

    s = jnp.sum(run_min)
    loss_ref[...] = jnp.broadcast_to(s.reshape(1, 1, 1), (1, 1, 128))
    idx_ref[...] = run_idx.reshape(1, _T, 1)
    fidx_ref[...] = (run_idx + h * _K).reshape(1, _T, 1)


@jax.jit
def _run_argmin(encoding, codebook):
    return pl.pallas_call(
        _argmin_head_kernel,
        grid=(_H,),
        in_specs=[
            pl.BlockSpec((_B, _S, _Dh), lambda h: (0, 0, h)),
            pl.BlockSpec((1, _K, _Dh), lambda h: (h, 0, 0)),
        ],
        out_specs=[
            pl.BlockSpec((1, _T, 1), lambda h: (h, 0, 0)),
            pl.BlockSpec((1, _T, 1), lambda h: (h, 0, 0)),
            pl.BlockSpec((1, 1, 128), lambda h: (h, 0, 0)),
        ],
        out_shape=[
            jax.ShapeDtypeStruct((_H, _T, 1), jnp.int32),
            jax.ShapeDtypeStruct((_H, _T, 1), jnp.int32),
            jax.ShapeDtypeStruct((_H, 1, 128), jnp.float32),
        ],
    )(encoding, codebook)


def _sc_gather_body(table_hbm, idx_hbm, out_hbm, idx_v, rows_v, sem0, sem1):
    nc = 2
    wid = lax.axis_index("s") * nc + lax.axis_index("c")
    base = wid * _BPW
    pltpu.sync_copy(idx_hbm.at[wid], idx_v)
    c0 = pltpu.async_copy(
        table_hbm.at[idx_v.at[0]], rows_v.at[pl.ds(0, 128)], sem0)
    c1 = pltpu.async_copy(
        table_hbm.at[idx_v.at[1]], rows_v.at[pl.ds(128, 128)], sem1)
    c0.wait()
    c1.wait()
    pltpu.sync_copy(rows_v, out_hbm.at[pl.ds(base, _BPW)])


@jax.jit
def _run_gather(table, idx_flat):
    mesh = plsc.VectorSubcoreMesh(
        core_axis_name="c", subcore_axis_name="s",
        num_cores=2, num_subcores=16)
    return pl.kernel(
        _sc_gather_body,
        out_type=jax.ShapeDtypeStruct((_TPAD, _Dh), jnp.float32),
        mesh=mesh,
        scratch_types=[
            pltpu.VMEM((2, 128), jnp.int32),
            pltpu.VMEM((_BPW, _Dh), jnp.float32),
            pltpu.SemaphoreType.DMA,
            pltpu.SemaphoreType.DMA,
        ],
    )(table, idx_flat.reshape(_NW, 2, 128))


def kernel(encoding, codebook, global_step):
    idx, fidx, loss_parts = _run_argmin(encoding, codebook)
    idx_sc = fidx[:, :, 0].T.reshape(_H * _T)
    idx_sc = jnp.concatenate(
        [idx_sc, jnp.zeros((_TPAD - _H * _T,), jnp.int32)])
    q_rows = _run_gather(codebook.reshape(_H * _K, _Dh), idx_sc)
    encoding_post = q_rows[:_H * _T].reshape(_B, _S, _D)
    vq_loss = 1.25 * jnp.sum(loss_parts[:, 0, 0]) / (_H * _T * _Dh)
    step = jnp.asarray(global_step).astype(vq_loss.dtype)
    memory_loss = vq_loss + 0.0 * step
    vq_codes = idx[:, :, 0].reshape(_H, _B, _S).transpose(1, 0, 2)
    return encoding_post, encoding, memory_loss, vq_codes


# speedup vs baseline: 1.3604x; 1.3604x over previous
import functools

import jax
import jax.numpy as jnp
from jax.experimental import pallas as pl


_B, _S, _D = 4, 196, 1024
_H, _K, _Dh = 8, 8192, 128
_T = _B * _S
_KC = 1024
_NKC = _K // _KC


def _vq_head_kernel(enc_ref, cb_ref, q_ref, idx_ref, loss_ref):
    z = enc_ref[...].reshape(_T, _Dh)
    cb = cb_ref[0]

    z_sq = jnp.sum(z * z, axis=1, keepdims=True)
    zm2 = -2.0 * z

    run_min = jnp.full((_T, 1), jnp.inf, dtype=jnp.float32)
    run_idx = jnp.zeros((_T, 1), dtype=jnp.int32)
    iota = jax.lax.broadcasted_iota(jnp.int32, (_T, _KC), 1)

    for kb in range(_NKC):
        cbk = cb[kb * _KC:(kb + 1) * _KC, :]
        dots2 = jax.lax.dot_general(
            zm2, cbk, (((1,), (1,)), ((), ())),
            preferred_element_type=jnp.float32)
        csq = jnp.sum(cbk * cbk, axis=1)[None, :]
        d = (z_sq + dots2) + csq
        mv = jnp.min(d, axis=1, keepdims=True)
        cand = jnp.where(d == mv, iota, _K)
        li = jnp.min(cand, axis=1, keepdims=True).astype(jnp.int32)
        better = mv < run_min
        run_idx = jnp.where(better, li + kb * _KC, run_idx)
        run_min = jnp.where(better, mv, run_min)

    q = jnp.zeros((_T, _Dh), dtype=jnp.float32)
    for kb in range(_NKC):
        cbk = cb[kb * _KC:(kb + 1) * _KC, :]
        rel = run_idx - kb * _KC
        oh = (iota == rel).astype(jnp.float32)
        q = q + jax.lax.dot_general(
            oh, cbk, (((1,), (0,)), ((), ())),
            preferred_element_type=jnp.float32)

    s = jnp.sum(run_min)
    loss_ref[...] = jnp.broadcast_to(s.reshape(1, 1, 1), (1, 1, 128))
    idx_ref[...] = run_idx.reshape(1, _T, 1)
    q_ref[...] = q.reshape(_B, _S, _Dh)


@jax.jit
def _run(encoding, codebook):
    q, idx, loss = pl.pallas_call(
        _vq_head_kernel,
        grid=(_H,),
        in_specs=[
            pl.BlockSpec((_B, _S, _Dh), lambda h: (0, 0, h)),
            pl.BlockSpec((1, _K, _Dh), lambda h: (h, 0, 0)),
        ],
        out_specs=[
            pl.BlockSpec((_B, _S, _Dh), lambda h: (0, 0, h)),
            pl.BlockSpec((1, _T, 1), lambda h: (h, 0, 0)),
            pl.BlockSpec((1, 1, 128), lambda h: (h, 0, 0)),
        ],
        out_shape=[
            jax.ShapeDtypeStruct((_B, _S, _D), jnp.float32),
            jax.ShapeDtypeStruct((_H, _T, 1), jnp.int32),
            jax.ShapeDtypeStruct((_H, 1, 128), jnp.float32),
        ],
    )(encoding, codebook)
    return q, idx, loss


def kernel(encoding, codebook, global_step):
    encoding_post, idx, loss_parts = _run(encoding, codebook)
    vq_loss = 1.25 * jnp.sum(loss_parts[:, 0, 0]) / (_H * _T * _Dh)
    step = jnp.asarray(global_step).astype(vq_loss.dtype)
    memory_loss = vq_loss + 0.0 * step
    vq_codes = idx[:, :, 0].reshape(_H, _B, _S).transpose(1, 0, 2)
    return encoding_post, encoding, memory_loss, vq_codes
